# baseline (speedup 1.0000x reference)
import functools

import jax
import jax.numpy as jnp
from jax import lax
from jax.experimental import pallas as pl
from jax.experimental.pallas import tpu as pltpu
from jax.experimental.pallas import tpu_sc as plsc

VOCAB = 1000000
EMBED_DIM = 64
BATCH = 16384
FIELDS = 26

NC = 2
NS = 16
NW = NC * NS

TOTAL = BATCH * FIELDS
PER_W = TOTAL // NW
CHUNK = 128
NCHUNK = PER_W // CHUNK


def _body(weight_hbm, idx_hbm, out_hbm, idx_v, rows_v, sem_g):
    wid = lax.axis_index("s") * NC + lax.axis_index("c")
    base = wid * PER_W
    pltpu.sync_copy(idx_hbm.at[wid], idx_v)

    def step(j, carry):
        pltpu.async_copy(weight_hbm.at[idx_v.at[j]], rows_v, sem_g).wait()
        pltpu.sync_copy(rows_v, out_hbm.at[pl.ds(base + j * CHUNK, CHUNK)])
        return carry

    lax.fori_loop(0, NCHUNK, step, 0)


@jax.jit
def _embed(idx, weight):
    mesh = plsc.VectorSubcoreMesh(core_axis_name="c", subcore_axis_name="s")
    k = pl.kernel(
        _body,
        out_type=jax.ShapeDtypeStruct((TOTAL, EMBED_DIM), jnp.float32),
        mesh=mesh,
        scratch_types=[
            pltpu.VMEM((NCHUNK, CHUNK), jnp.int32),
            pltpu.VMEM((CHUNK, EMBED_DIM), jnp.float32),
            pltpu.SemaphoreType.DMA,
        ],
        compiler_params=pltpu.CompilerParams(use_tc_tiling_on_sc=False),
    )
    return k(weight, idx)


def kernel(input, weight):
    idx = input.astype(jnp.int32).reshape(NW, NCHUNK, CHUNK)
    out = _embed(idx, weight)
    return out.reshape(BATCH, FIELDS, EMBED_DIM)


# speedup vs baseline: 1.0771x; 1.0771x over previous
import functools

import jax
import jax.numpy as jnp
from jax import lax
from jax.experimental import pallas as pl
from jax.experimental.pallas import tpu as pltpu
from jax.experimental.pallas import tpu_sc as plsc

VOCAB = 1000000
EMBED_DIM = 64
BATCH = 16384
FIELDS = 26

NC = 2
NS = 16
NW = NC * NS

TOTAL = BATCH * FIELDS
PER_W = TOTAL // NW
CHUNK = 128
NCHUNK = PER_W // CHUNK


NBUF = 4
NGROUP = NCHUNK // NBUF


def _body(weight_hbm, idx_hbm, out_hbm, idx_v, *scr):
    bufs = scr[:NBUF]
    semg = scr[NBUF:]
    wid = lax.axis_index("s") * NC + lax.axis_index("c")
    base = wid * PER_W
    pltpu.sync_copy(idx_hbm.at[wid], idx_v)

    for c in range(NBUF):
        pltpu.async_copy(weight_hbm.at[idx_v.at[c]], bufs[c], semg[c])

    def group(g, carry):
        j0 = g * NBUF
        for c in range(NBUF):
            j = j0 + c
            pltpu.make_async_copy(weight_hbm.at[idx_v.at[j]], bufs[c], semg[c]).wait()
            pltpu.sync_copy(bufs[c], out_hbm.at[pl.ds(base + j * CHUNK, CHUNK)])
            pltpu.async_copy(weight_hbm.at[idx_v.at[j + NBUF]], bufs[c], semg[c])
        return carry

    lax.fori_loop(0, NGROUP - 1, group, 0)

    j0 = (NGROUP - 1) * NBUF
    for c in range(NBUF):
        j = j0 + c
        pltpu.make_async_copy(weight_hbm.at[idx_v.at[j]], bufs[c], semg[c]).wait()
        pltpu.sync_copy(bufs[c], out_hbm.at[pl.ds(base + j * CHUNK, CHUNK)])


@jax.jit
def _embed(idx, weight):
    mesh = plsc.VectorSubcoreMesh(core_axis_name="c", subcore_axis_name="s")
    k = pl.kernel(
        _body,
        out_type=jax.ShapeDtypeStruct((TOTAL, EMBED_DIM), jnp.float32),
        mesh=mesh,
        scratch_types=(
            [pltpu.VMEM((NCHUNK, CHUNK), jnp.int32)]
            + [pltpu.VMEM((CHUNK, EMBED_DIM), jnp.float32) for _ in range(NBUF)]
            + [pltpu.SemaphoreType.DMA for _ in range(NBUF)]
        ),
        compiler_params=pltpu.CompilerParams(use_tc_tiling_on_sc=False),
    )
    return k(weight, idx)


def kernel(input, weight):
    idx = input.astype(jnp.int32).reshape(NW, NCHUNK, CHUNK)
    out = _embed(idx, weight)
    return out.reshape(BATCH, FIELDS, EMBED_DIM)
